# baseline (speedup 1.0000x reference)
import functools
import math

import jax
import jax.numpy as jnp
from jax import lax
from jax.experimental import pallas as pl
from jax.experimental.pallas import tpu as pltpu
from jax.experimental.pallas import tpu_sc as plsc

_NBUF = 4
_INFLIGHT = 2


def _tab_transpose_body(scale, in_ref, out_ref):
    out_ref[...] = in_ref[...].T * scale


def _tc_tab_transpose(tab_t, scale):
    d, v = tab_t.shape
    cw = 8192
    return pl.pallas_call(
        functools.partial(_tab_transpose_body, float(scale)),
        grid=(pl.cdiv(v, cw),),
        in_specs=[pl.BlockSpec((d, cw), lambda j: (0, j))],
        out_specs=pl.BlockSpec((cw, d), lambda j: (j, 0)),
        out_shape=jax.ShapeDtypeStruct((v, d), jnp.float32),
    )(tab_t)


def _out_transpose_body(in_ref, out_ref):
    out_ref[...] = jnp.transpose(in_ref[...], (1, 2, 0))


def _tc_out_transpose(g3):
    b0, hist, d = g3.shape
    bb, hh = 512, 8
    return pl.pallas_call(
        _out_transpose_body,
        grid=(b0 // bb, hist // hh),
        in_specs=[pl.BlockSpec((bb, hh, d), lambda i, j: (i, j, 0))],
        out_specs=pl.BlockSpec((hh, d, bb), lambda i, j: (j, 0, i)),
        out_shape=jax.ShapeDtypeStruct((hist, d, b0), jnp.float32),
    )(g3)


def _gather_body(tok_hbm, tab_hbm, out_hbm, idx_v, rows_v, *sems,
                 n_chunks, chunk, b_per_w, nc):
    gsems = sems[:_NBUF]
    wsems = sems[_NBUF:]
    wid = lax.axis_index("s") * nc + lax.axis_index("c")
    base = wid * b_per_w

    def issue_gather(c, s):
        pltpu.sync_copy(tok_hbm.at[pl.ds(base + c * chunk, chunk)],
                        idx_v.at[s])
        pltpu.make_async_copy(tab_hbm.at[idx_v.at[s]], rows_v.at[s],
                              gsems[s]).start()

    for c0 in range(_INFLIGHT):
        issue_gather(c0, c0)

    n_outer = n_chunks // _NBUF

    def outer(o, carry):
        for s in range(_NBUF):
            c = o * _NBUF + s
            pltpu.make_async_copy(tab_hbm.at[idx_v.at[s]], rows_v.at[s],
                                  gsems[s]).wait()
            pltpu.make_async_copy(rows_v.at[s],
                                  out_hbm.at[pl.ds(base + c * chunk, chunk)],
                                  wsems[s]).start()

            cn = c + _INFLIGHT
            sn = (s + _INFLIGHT) % _NBUF

            @pl.when(cn < n_chunks)
            def _():
                @pl.when(cn >= _NBUF)
                def _():
                    pltpu.make_async_copy(
                        rows_v.at[sn], out_hbm.at[pl.ds(base, chunk)],
                        wsems[sn]).wait()
                issue_gather(cn, sn)
        return carry

    lax.fori_loop(0, n_outer, outer, 0)

    for s in range(_NBUF):
        pltpu.make_async_copy(rows_v.at[s], out_hbm.at[pl.ds(base, chunk)],
                              wsems[s]).wait()


def _sc_gather(flat_tokens, tab_rm):
    (b,) = flat_tokens.shape
    v, d = tab_rm.shape

    info = plsc.get_sparse_core_info()
    nc, ns = info.num_cores, info.num_subcores
    nw = nc * ns
    b_per_w = b // nw
    chunk = 800
    n_chunks = b_per_w // chunk
    assert b % nw == 0 and b_per_w % chunk == 0
    assert n_chunks % _NBUF == 0 and chunk % 8 == 0

    mesh = plsc.VectorSubcoreMesh(core_axis_name="c", subcore_axis_name="s")
    body = functools.partial(
        _gather_body, n_chunks=n_chunks, chunk=chunk, b_per_w=b_per_w, nc=nc)

    k = functools.partial(
        pl.kernel,
        mesh=mesh,
        compiler_params=pltpu.CompilerParams(use_tc_tiling_on_sc=False),
        out_type=jax.ShapeDtypeStruct((b, d), jnp.float32),
        scratch_types=[
            pltpu.VMEM((_NBUF, chunk), jnp.int32),
            pltpu.VMEM((_NBUF, chunk, d), jnp.float32),
        ] + [pltpu.SemaphoreType.DMA] * (2 * _NBUF),
    )(body)

    return k(flat_tokens, tab_rm)


def kernel(tokens, table):
    b0, hist = tokens.shape
    v, d = table.shape
    b = b0 * hist

    tab_rm = _tc_tab_transpose(table.T, math.sqrt(d))
    gathered = _sc_gather(tokens.reshape(b), tab_rm)
    out3 = _tc_out_transpose(gathered.reshape(b0, hist, d))
    return out3.transpose(2, 0, 1)


# speedup vs baseline: 1.9304x; 1.9304x over previous
import functools
import math

import jax
import jax.numpy as jnp
from jax import lax
from jax.experimental import pallas as pl
from jax.experimental.pallas import tpu as pltpu
from jax.experimental.pallas import tpu_sc as plsc

_NBUF = 4
_INFLIGHT = 2


def _tab_transpose_body(scale, d, in_ref, out_ref):
    out_ref[:, :d] = in_ref[...].T * scale


def _tc_tab_transpose(tab_t, scale):
    d, v = tab_t.shape
    cw = 2048
    return pl.pallas_call(
        functools.partial(_tab_transpose_body, float(scale), d),
        grid=(pl.cdiv(v, cw),),
        in_specs=[pl.BlockSpec((d, cw), lambda j: (0, j))],
        out_specs=pl.BlockSpec((cw, 128), lambda j: (j, 0)),
        out_shape=jax.ShapeDtypeStruct((v, 128), jnp.float32),
    )(tab_t)


def _gather_body(tok_hbm, tab_hbm, out_hbm, idx_v, rows_v, *sems,
                 n_chunks, chunk, b_per_w, nc):
    gsems = sems[:_NBUF]
    wsems = sems[_NBUF:]
    wid = lax.axis_index("s") * nc + lax.axis_index("c")
    base = wid * b_per_w

    def issue_gather(c, s):
        pltpu.sync_copy(tok_hbm.at[pl.ds(base + c * chunk, chunk)],
                        idx_v.at[s])
        pltpu.make_async_copy(tab_hbm.at[idx_v.at[s]], rows_v.at[s],
                              gsems[s]).start()

    for c0 in range(_INFLIGHT):
        issue_gather(c0, c0)

    n_outer = n_chunks // _NBUF

    def outer(o, carry):
        for s in range(_NBUF):
            c = o * _NBUF + s
            pltpu.make_async_copy(tab_hbm.at[idx_v.at[s]], rows_v.at[s],
                                  gsems[s]).wait()
            pltpu.make_async_copy(rows_v.at[s],
                                  out_hbm.at[pl.ds(base + c * chunk, chunk)],
                                  wsems[s]).start()

            cn = c + _INFLIGHT
            sn = (s + _INFLIGHT) % _NBUF

            @pl.when(cn < n_chunks)
            def _():
                @pl.when(cn >= _NBUF)
                def _():
                    pltpu.make_async_copy(
                        rows_v.at[sn], out_hbm.at[pl.ds(base, chunk)],
                        wsems[sn]).wait()
                issue_gather(cn, sn)
        return carry

    lax.fori_loop(0, n_outer, outer, 0)

    for s in range(_NBUF):
        pltpu.make_async_copy(rows_v.at[s], out_hbm.at[pl.ds(base, chunk)],
                              wsems[s]).wait()


def _sc_gather(flat_tokens, tab_rm):
    (b,) = flat_tokens.shape
    v, d = tab_rm.shape

    info = plsc.get_sparse_core_info()
    nc, ns = info.num_cores, info.num_subcores
    nw = nc * ns
    b_per_w = b // nw
    chunk = 800
    n_chunks = b_per_w // chunk
    assert b % nw == 0 and b_per_w % chunk == 0
    assert n_chunks % _NBUF == 0 and chunk % 8 == 0

    mesh = plsc.VectorSubcoreMesh(core_axis_name="c", subcore_axis_name="s")
    body = functools.partial(
        _gather_body, n_chunks=n_chunks, chunk=chunk, b_per_w=b_per_w, nc=nc)

    k = functools.partial(
        pl.kernel,
        mesh=mesh,
        compiler_params=pltpu.CompilerParams(use_tc_tiling_on_sc=False),
        out_type=jax.ShapeDtypeStruct((b, d), jnp.float32),
        scratch_types=[
            pltpu.VMEM((_NBUF, chunk), jnp.int32),
            pltpu.VMEM((_NBUF, chunk, d), jnp.float32),
        ] + [pltpu.SemaphoreType.DMA] * (2 * _NBUF),
    )(body)

    return k(flat_tokens, tab_rm)


def kernel(tokens, table):
    b0, hist = tokens.shape
    v, d = table.shape
    b = b0 * hist

    stride = 128 // d
    tab_padded = _tc_tab_transpose(table.T, math.sqrt(d))
    tab_rm = tab_padded.reshape(v * stride, d)
    gathered = _sc_gather(tokens.reshape(b) * stride, tab_rm)
    return gathered.reshape(b0, hist, d)


# speedup vs baseline: 3.3009x; 1.7100x over previous
import functools
import math

import jax
import jax.numpy as jnp
from jax import lax
from jax.experimental import pallas as pl
from jax.experimental.pallas import tpu as pltpu
from jax.experimental.pallas import tpu_sc as plsc

_NBUF = 4
_INFLIGHT = 2


def _tab_transpose_body(scale, d, in_ref, out_ref):
    out_ref[:, :d] = in_ref[...].T * scale


def _tc_tab_transpose(tab_t, scale):
    d, v = tab_t.shape
    cw = 4096
    return pl.pallas_call(
        functools.partial(_tab_transpose_body, float(scale), d),
        grid=(pl.cdiv(v, cw),),
        in_specs=[pl.BlockSpec((d, cw), lambda j: (0, j))],
        out_specs=pl.BlockSpec((cw, 128), lambda j: (j, 0)),
        out_shape=jax.ShapeDtypeStruct((v, 128), jnp.float32),
    )(tab_t)


def _gather_body(tok_hbm, tab_hbm, out_hbm, idx_v, rows_v, *sems,
                 n_chunks, chunk, b_per_w, nc, d):
    gsems = sems[:_NBUF]
    wsems = sems[_NBUF:]
    wid = lax.axis_index("s") * nc + lax.axis_index("c")
    base = wid * b_per_w

    def issue_gather(c, s):
        pltpu.sync_copy(tok_hbm.at[pl.ds(base + c * chunk, chunk)],
                        idx_v.at[s])
        pltpu.make_async_copy(tab_hbm.at[idx_v.at[s]], rows_v.at[s],
                              gsems[s]).start()

    for c0 in range(_INFLIGHT):
        issue_gather(c0, c0)

    n_outer = n_chunks // _NBUF

    def outer(o, carry):
        for s in range(_NBUF):
            c = o * _NBUF + s
            pltpu.make_async_copy(tab_hbm.at[idx_v.at[s]], rows_v.at[s],
                                  gsems[s]).wait()
            pltpu.make_async_copy(
                rows_v.at[s],
                out_hbm.at[pl.ds(base + c * chunk, chunk), pl.ds(0, d)],
                wsems[s]).start()

            cn = c + _INFLIGHT
            sn = (s + _INFLIGHT) % _NBUF

            @pl.when(cn < n_chunks)
            def _():
                @pl.when(cn >= _NBUF)
                def _():
                    pltpu.make_async_copy(
                        rows_v.at[sn],
                        out_hbm.at[pl.ds(base, chunk), pl.ds(0, d)],
                        wsems[sn]).wait()
                issue_gather(cn, sn)
        return carry

    lax.fori_loop(0, n_outer, outer, 0)

    for s in range(_NBUF):
        pltpu.make_async_copy(rows_v.at[s],
                              out_hbm.at[pl.ds(base, chunk), pl.ds(0, d)],
                              wsems[s]).wait()


def _sc_gather(flat_tokens, tab_rm):
    (b,) = flat_tokens.shape
    v, d = tab_rm.shape

    info = plsc.get_sparse_core_info()
    nc, ns = info.num_cores, info.num_subcores
    nw = nc * ns
    b_per_w = b // nw
    chunk = 800
    n_chunks = b_per_w // chunk
    assert b % nw == 0 and b_per_w % chunk == 0
    assert n_chunks % _NBUF == 0 and chunk % 8 == 0

    mesh = plsc.VectorSubcoreMesh(core_axis_name="c", subcore_axis_name="s")
    body = functools.partial(
        _gather_body, n_chunks=n_chunks, chunk=chunk, b_per_w=b_per_w,
        nc=nc, d=d)

    k = functools.partial(
        pl.kernel,
        mesh=mesh,
        compiler_params=pltpu.CompilerParams(use_tc_tiling_on_sc=False),
        out_type=jax.ShapeDtypeStruct((b, 128), jnp.float32),
        scratch_types=[
            pltpu.VMEM((_NBUF, chunk), jnp.int32),
            pltpu.VMEM((_NBUF, chunk, d), jnp.float32),
        ] + [pltpu.SemaphoreType.DMA] * (2 * _NBUF),
    )(body)

    return k(flat_tokens, tab_rm)


def kernel(tokens, table):
    b0, hist = tokens.shape
    v, d = table.shape
    b = b0 * hist

    stride = 128 // d
    tab_padded = _tc_tab_transpose(table.T, math.sqrt(d))
    tab_rm = tab_padded.reshape(v * stride, d)
    gathered = _sc_gather(tokens.reshape(b) * stride, tab_rm)
    return gathered.reshape(b0, hist, 128)[:, :, :d]
